# Initial kernel scaffold; baseline (speedup 1.0000x reference)
import jax
import jax.numpy as jnp
from jax.experimental import pallas as pl


def kernel(x, classifier_params, net1_params, net2_params, net3_params):
    raise NotImplementedError("write your pallas kernel here")


# speedup vs baseline: 2.2782x; 2.2782x over previous
import functools

import jax
import jax.numpy as jnp
from jax.experimental import pallas as pl
from jax.experimental.pallas import tpu as pltpu

_SUBNET_CAPS = (27, 50, 28)
_NG = 2
_NB = 2


def _rowshift(a, o):
    if o == 0:
        return a
    z = jnp.zeros((abs(o), a.shape[1]), a.dtype)
    if o > 0:
        return jnp.concatenate([a[o:], z], axis=0)
    return jnp.concatenate([z, a[:o]], axis=0)


def _im2col(x, h, w):
    c = x.shape[1]
    p = jnp.concatenate([_rowshift(x, (dy - 1) * w) for dy in (0, 1, 2)],
                        axis=1)
    xmod = jax.lax.broadcasted_iota(jnp.int32, (h * w, 1), 0) % w
    s0 = jnp.where(xmod != 0, _rowshift(p, -1), 0.0)
    s2 = jnp.where(xmod != w - 1, _rowshift(p, 1), 0.0)
    return jnp.concatenate([s0, p, s2], axis=1)


def _conv3(x, h, w, wmat, bias):
    pat = _im2col(x, h, w)
    return jnp.dot(pat, wmat, preferred_element_type=jnp.float32) + bias


def _ps_xla(v, h):
    n, _, c4 = v.shape
    c = c4 // 4
    v = v.reshape(n, h, h, c, 2, 2).transpose(0, 1, 4, 2, 5, 3)
    return v.reshape(n, 4 * h * h, c)


def _classifier_route_kernel(xc_ref, w1_ref, b1_ref, w2_ref, b2_ref,
                             fw_ref, fb_ref,
                             pad0_ref, pad1_ref, pad2_ref, inv_ref, takes_ref):
    b = xc_ref.shape[0]
    npos = xc_ref.shape[1]
    xc = xc_ref[...].reshape(b * npos, xc_ref.shape[2]) * (1.0 / 255.0)
    h = jnp.dot(xc, w1_ref[...], preferred_element_type=jnp.float32) + b1_ref[...]
    h = jax.nn.leaky_relu(h, 0.1)
    h = jnp.dot(h, w2_ref[...], preferred_element_type=jnp.float32) + b2_ref[...]
    h = jax.nn.leaky_relu(h, 0.1)
    pooled = jnp.mean(h.reshape(b, npos, h.shape[1]), axis=1)
    logits = jnp.dot(pooled, fw_ref[...], preferred_element_type=jnp.float32) + fb_ref[...]

    lt = logits.T
    l0, l1, l2 = lt[0:1, :], lt[1:2, :], lt[2:3, :]
    a_row = jnp.where((l0 >= l1) & (l0 >= l2), 0,
                      jnp.where(l1 >= l2, 1, 2)).astype(jnp.int32)
    a_col = a_row.T

    cmp_le = (jax.lax.broadcasted_iota(jnp.int32, (b, b), 0)
              <= jax.lax.broadcasted_iota(jnp.int32, (b, b), 1))
    i_row = jax.lax.broadcasted_iota(jnp.int32, (1, b), 1)

    pad_refs = (pad0_ref, pad1_ref, pad2_ref)
    safe_cols = []
    take_vals = []
    for e, cap in enumerate(_SUBNET_CAPS):
        m_col = a_col == e
        m_row = a_row == e
        ranks_row = jnp.sum(jnp.where(cmp_le & m_col, 1, 0), axis=0, keepdims=True)
        cnt = jnp.sum(m_col.astype(jnp.int32))
        take = jnp.minimum(cnt, jnp.int32(cap))
        j_col = jax.lax.broadcasted_iota(jnp.int32, (cap, b), 0)
        sel = (ranks_row == (j_col + 1)) & m_row
        pad_col = jnp.sum(jnp.where(sel, i_row, 0), axis=1, keepdims=True)
        jj = jax.lax.broadcasted_iota(jnp.int32, (cap, 1), 0)
        safe_col = jnp.where(jj < take, pad_col, jnp.int32(b))
        pad_refs[e][...] = pad_col
        safe_cols.append(safe_col)
        take_vals.append(take.reshape(1, 1))

    safe_all = jnp.concatenate(safe_cols, axis=0)
    safe_row = safe_all.T
    r_col = jax.lax.broadcasted_iota(jnp.int32, (b, b), 0)
    k_row = jax.lax.broadcasted_iota(jnp.int32, (b, b), 1)
    eq = safe_row == r_col
    hit = jnp.any(eq, axis=1, keepdims=True)
    inv = jnp.where(hit, jnp.sum(jnp.where(eq, k_row, 0), axis=1, keepdims=True),
                    jnp.int32(b))
    inv_ref[...] = inv
    takes_ref[...] = jnp.concatenate(take_vals, axis=0)


def _rcan_body_kernel(idx_ref, x_ref, *rest):
    del idx_ref
    o_ref = rest[-1]
    it = iter(rest[:-1])

    def nxt():
        return next(it)[...], next(it)[...]

    x0 = x_ref[0].reshape(32 * 32, 3)
    h = _conv3(x0, 32, 32, *nxt())
    body = h
    for _ in range(_NG):
        gin = body
        for _ in range(_NB):
            w1, b1 = nxt()
            w2, b2 = nxt()
            wa1, ba1 = nxt()
            wa2, ba2 = nxt()
            r = _conv3(jax.nn.relu(_conv3(body, 32, 32, w1, b1)), 32, 32, w2, b2)
            ca = jnp.mean(r, axis=0, keepdims=True)
            ca = jax.nn.relu(jnp.dot(ca, wa1, preferred_element_type=jnp.float32) + ba1)
            ca = jax.nn.sigmoid(jnp.dot(ca, wa2, preferred_element_type=jnp.float32) + ba2)
            body = body + r * ca
        wg, bg = nxt()
        body = _conv3(body, 32, 32, wg, bg) + gin
    wb, bb = nxt()
    body = _conv3(body, 32, 32, wb, bb) + h
    wu1, bu1 = nxt()
    o_ref[0] = _conv3(body, 32, 32, wu1, bu1)


def _up2_kernel(u_ref, w_ref, b_ref, o_ref):
    o_ref[0] = _conv3(u_ref[0], 64, 64, w_ref[...], b_ref[...])


def _tail_kernel(u_ref, w_ref, b_ref, o_ref):
    u = u_ref[0]
    wt = w_ref[...]
    bt = b_ref[...]
    c = u.shape[1]
    w = 128
    p = jnp.concatenate([_rowshift(u, (dy - 1) * w) for dy in (0, 1, 2)],
                        axis=1)
    nch = 4096
    xmod = jax.lax.broadcasted_iota(jnp.int32, (nch, 1), 0) % w
    zrow = jnp.zeros((1, 3 * c), jnp.float32)
    for ci in range(4):
        r0 = ci * nch
        pc = p[r0:r0 + nch]
        prev = p[r0 - 1:r0] if r0 > 0 else zrow
        nxt_row = p[r0 + nch:r0 + nch + 1] if r0 + nch < 16384 else zrow
        s0 = jnp.where(xmod != 0,
                       jnp.concatenate([prev, pc[:-1]], axis=0), 0.0)
        s2 = jnp.where(xmod != w - 1,
                       jnp.concatenate([pc[1:], nxt_row], axis=0), 0.0)
        pat = jnp.concatenate([s0, pc, s2], axis=1)
        out = jnp.dot(pat, wt, preferred_element_type=jnp.float32) + bt
        o_ref[0, :, 32 * ci:32 * (ci + 1), :] = out.T.reshape(3, 32, 128)


def _combine_kernel(inv_ref, y_ref, o_ref):
    del inv_ref
    o_ref[...] = y_ref[...]


def _wprep(cp, k):
    w = cp['w']
    cout, cin = w.shape[0], w.shape[1]
    if k == 3:
        w2 = jnp.transpose(w, (3, 2, 1, 0)).reshape(9 * cin, cout)
    else:
        w2 = w.reshape(cout, cin).T
    return [w2, cp['b'].reshape(1, cout)]


def _flat_rcan_body(p):
    seq = []
    seq += _wprep(p['head'], 3)
    for g in p['groups']:
        for blk in g['blocks']:
            seq += _wprep(blk['c1'], 3)
            seq += _wprep(blk['c2'], 3)
            seq += _wprep(blk['ca1'], 1)
            seq += _wprep(blk['ca2'], 1)
        seq += _wprep(g['conv'], 3)
    seq += _wprep(p['body_conv'], 3)
    seq += _wprep(p['up1'], 3)
    return seq


def _run_subnet(pad_idx, x_nhwc, params, cap):
    nf = params['head']['w'].shape[0]
    flat_w = _flat_rcan_body(params)
    in_specs = [pl.BlockSpec((1, 32, 32, 3), lambda i, idx: (idx[i], 0, 0, 0))]
    for wa in flat_w:
        in_specs.append(pl.BlockSpec(wa.shape,
                                     functools.partial(lambda n, i, idx: (0,) * n,
                                                       wa.ndim)))
    va = pl.pallas_call(
        _rcan_body_kernel,
        grid_spec=pltpu.PrefetchScalarGridSpec(
            num_scalar_prefetch=1,
            grid=(cap,),
            in_specs=in_specs,
            out_specs=pl.BlockSpec((1, 1024, 4 * nf), lambda i, idx: (i, 0, 0)),
        ),
        out_shape=jax.ShapeDtypeStruct((cap, 1024, 4 * nf), jnp.float32),
    )(pad_idx, x_nhwc, *flat_w)

    u1 = _ps_xla(va, 32)
    wu2, bu2 = _wprep(params['up2'], 3)
    vb = pl.pallas_call(
        _up2_kernel,
        grid=(cap,),
        in_specs=[pl.BlockSpec((1, 4096, nf), lambda i: (i, 0, 0)),
                  pl.BlockSpec(wu2.shape, lambda i: (0, 0)),
                  pl.BlockSpec(bu2.shape, lambda i: (0, 0))],
        out_specs=pl.BlockSpec((1, 4096, 4 * nf), lambda i: (i, 0, 0)),
        out_shape=jax.ShapeDtypeStruct((cap, 4096, 4 * nf), jnp.float32),
    )(u1, wu2, bu2)

    u2 = _ps_xla(vb, 64)
    wt, bt = _wprep(params['tail'], 3)
    return pl.pallas_call(
        _tail_kernel,
        grid=(cap,),
        in_specs=[pl.BlockSpec((1, 16384, nf), lambda i: (i, 0, 0)),
                  pl.BlockSpec(wt.shape, lambda i: (0, 0)),
                  pl.BlockSpec(bt.shape, lambda i: (0, 0))],
        out_specs=pl.BlockSpec((1, 3, 128, 128), lambda i: (i, 0, 0, 0)),
        out_shape=jax.ShapeDtypeStruct((cap, 3, 128, 128), jnp.float32),
    )(u2, wt, bt)


def kernel(x, classifier_params, net1_params, net2_params, net3_params):
    b = x.shape[0]

    xc = x.reshape(b, 3, 8, 4, 8, 4).transpose(0, 2, 4, 1, 3, 5).reshape(b, 64, 48)
    cp = classifier_params
    cw1 = jnp.transpose(cp['c1']['w'], (1, 2, 3, 0)).reshape(48, 128)
    cb1 = cp['c1']['b'].reshape(1, 128)
    cw2 = cp['c2']['w'].reshape(128, 128).T
    cb2 = cp['c2']['b'].reshape(1, 128)
    fw = cp['fc_w']
    fb = cp['fc_b'].reshape(1, 3)

    pad0, pad1, pad2, inv, takes = pl.pallas_call(
        _classifier_route_kernel,
        out_shape=[
            jax.ShapeDtypeStruct((_SUBNET_CAPS[0], 1), jnp.int32),
            jax.ShapeDtypeStruct((_SUBNET_CAPS[1], 1), jnp.int32),
            jax.ShapeDtypeStruct((_SUBNET_CAPS[2], 1), jnp.int32),
            jax.ShapeDtypeStruct((b, 1), jnp.int32),
            jax.ShapeDtypeStruct((3, 1), jnp.int32),
        ],
    )(xc, cw1, cb1, cw2, cb2, fw, fb)

    x_nhwc = jnp.transpose(x, (0, 2, 3, 1))
    ys = []
    for pad_e, params, cap in ((pad0, net1_params, _SUBNET_CAPS[0]),
                               (pad1, net2_params, _SUBNET_CAPS[1]),
                               (pad2, net3_params, _SUBNET_CAPS[2])):
        ys.append(_run_subnet(pad_e.reshape(cap), x_nhwc, params, cap))

    y_ext = jnp.concatenate(ys + [jnp.zeros((1, 3, 128, 128), jnp.float32)], axis=0)
    outs = pl.pallas_call(
        _combine_kernel,
        grid_spec=pltpu.PrefetchScalarGridSpec(
            num_scalar_prefetch=1,
            grid=(b,),
            in_specs=[pl.BlockSpec((1, 3, 128, 128), lambda i, inv: (inv[i], 0, 0, 0))],
            out_specs=pl.BlockSpec((1, 3, 128, 128), lambda i, inv: (i, 0, 0, 0)),
        ),
        out_shape=jax.ShapeDtypeStruct((b, 3, 128, 128), jnp.float32),
    )(inv.reshape(b), y_ext)

    return outs, takes.reshape(3)


# speedup vs baseline: 2.2811x; 1.0013x over previous
import functools

import jax
import jax.numpy as jnp
from jax.experimental import pallas as pl
from jax.experimental.pallas import tpu as pltpu

_SUBNET_CAPS = (27, 50, 28)
_NG = 2
_NB = 2


def _rowshift(a, o):
    if o == 0:
        return a
    z = jnp.zeros((abs(o), a.shape[1]), a.dtype)
    if o > 0:
        return jnp.concatenate([a[o:], z], axis=0)
    return jnp.concatenate([z, a[:o]], axis=0)


def _im2col(x, h, w):
    c = x.shape[1]
    p = jnp.concatenate([_rowshift(x, (dy - 1) * w) for dy in (0, 1, 2)],
                        axis=1)
    xmod = jax.lax.broadcasted_iota(jnp.int32, (h * w, 1), 0) % w
    s0 = jnp.where(xmod != 0, _rowshift(p, -1), 0.0)
    s2 = jnp.where(xmod != w - 1, _rowshift(p, 1), 0.0)
    return jnp.concatenate([s0, p, s2], axis=1)


def _conv3(x, h, w, wmat, bias):
    pat = _im2col(x, h, w)
    return jnp.dot(pat, wmat, preferred_element_type=jnp.float32) + bias


def _ps_xla(v, h):
    n, _, c4 = v.shape
    c = c4 // 4
    v = v.reshape(n, h, h, c, 2, 2).transpose(0, 1, 4, 2, 5, 3)
    return v.reshape(n, 4 * h * h, c)


def _classifier_route_kernel(xc_ref, w1_ref, b1_ref, w2_ref, b2_ref,
                             fw_ref, fb_ref,
                             pad0_ref, pad1_ref, pad2_ref, inv_ref, takes_ref):
    b = xc_ref.shape[0]
    npos = xc_ref.shape[1]
    xc = xc_ref[...].reshape(b * npos, xc_ref.shape[2]) * (1.0 / 255.0)
    h = jnp.dot(xc, w1_ref[...], preferred_element_type=jnp.float32) + b1_ref[...]
    h = jax.nn.leaky_relu(h, 0.1)
    h = jnp.dot(h, w2_ref[...], preferred_element_type=jnp.float32) + b2_ref[...]
    h = jax.nn.leaky_relu(h, 0.1)
    pooled = jnp.mean(h.reshape(b, npos, h.shape[1]), axis=1)
    logits = jnp.dot(pooled, fw_ref[...], preferred_element_type=jnp.float32) + fb_ref[...]

    lt = logits.T
    l0, l1, l2 = lt[0:1, :], lt[1:2, :], lt[2:3, :]
    a_row = jnp.where((l0 >= l1) & (l0 >= l2), 0,
                      jnp.where(l1 >= l2, 1, 2)).astype(jnp.int32)
    a_col = a_row.T

    cmp_le = (jax.lax.broadcasted_iota(jnp.int32, (b, b), 0)
              <= jax.lax.broadcasted_iota(jnp.int32, (b, b), 1))
    i_row = jax.lax.broadcasted_iota(jnp.int32, (1, b), 1)

    pad_refs = (pad0_ref, pad1_ref, pad2_ref)
    safe_cols = []
    take_vals = []
    for e, cap in enumerate(_SUBNET_CAPS):
        m_col = a_col == e
        m_row = a_row == e
        ranks_row = jnp.sum(jnp.where(cmp_le & m_col, 1, 0), axis=0, keepdims=True)
        cnt = jnp.sum(m_col.astype(jnp.int32))
        take = jnp.minimum(cnt, jnp.int32(cap))
        j_col = jax.lax.broadcasted_iota(jnp.int32, (cap, b), 0)
        sel = (ranks_row == (j_col + 1)) & m_row
        pad_col = jnp.sum(jnp.where(sel, i_row, 0), axis=1, keepdims=True)
        jj = jax.lax.broadcasted_iota(jnp.int32, (cap, 1), 0)
        safe_col = jnp.where(jj < take, pad_col, jnp.int32(b))
        pad_refs[e][...] = pad_col
        safe_cols.append(safe_col)
        take_vals.append(take.reshape(1, 1))

    safe_all = jnp.concatenate(safe_cols, axis=0)
    safe_row = safe_all.T
    r_col = jax.lax.broadcasted_iota(jnp.int32, (b, b), 0)
    k_row = jax.lax.broadcasted_iota(jnp.int32, (b, b), 1)
    eq = safe_row == r_col
    hit = jnp.any(eq, axis=1, keepdims=True)
    inv = jnp.where(hit, jnp.sum(jnp.where(eq, k_row, 0), axis=1, keepdims=True),
                    jnp.int32(b))
    inv_ref[...] = inv
    takes_ref[...] = jnp.concatenate(take_vals, axis=0)


def _rcan_body_kernel(idx_ref, x_ref, *rest):
    del idx_ref
    o_ref = rest[-1]
    it = iter(rest[:-1])

    def nxt():
        return next(it)[...], next(it)[...]

    x0 = x_ref[0].reshape(32 * 32, 3)
    h = _conv3(x0, 32, 32, *nxt())
    body = h
    for _ in range(_NG):
        gin = body
        for _ in range(_NB):
            w1, b1 = nxt()
            w2, b2 = nxt()
            wa1, ba1 = nxt()
            wa2, ba2 = nxt()
            r = _conv3(jax.nn.relu(_conv3(body, 32, 32, w1, b1)), 32, 32, w2, b2)
            ca = jnp.mean(r, axis=0, keepdims=True)
            ca = jax.nn.relu(jnp.dot(ca, wa1, preferred_element_type=jnp.float32) + ba1)
            ca = jax.nn.sigmoid(jnp.dot(ca, wa2, preferred_element_type=jnp.float32) + ba2)
            body = body + r * ca
        wg, bg = nxt()
        body = _conv3(body, 32, 32, wg, bg) + gin
    wb, bb = nxt()
    body = _conv3(body, 32, 32, wb, bb) + h
    wu1, bu1 = nxt()
    o_ref[0] = _conv3(body, 32, 32, wu1, bu1)


def _up2_kernel(u_ref, w_ref, b_ref, o_ref):
    o_ref[0] = _conv3(u_ref[0], 64, 64, w_ref[...], b_ref[...])


def _tail_kernel(u_ref, w_ref, b_ref, o_ref):
    u = u_ref[0]
    wt = w_ref[...]
    bt = b_ref[...]
    c = u.shape[1]
    w = 128
    p = jnp.concatenate([_rowshift(u, (dy - 1) * w) for dy in (0, 1, 2)],
                        axis=1)
    nch = 4096
    xmod = jax.lax.broadcasted_iota(jnp.int32, (nch, 1), 0) % w
    zrow = jnp.zeros((1, 3 * c), jnp.float32)
    for ci in range(4):
        r0 = ci * nch
        pc = p[r0:r0 + nch]
        prev = p[r0 - 1:r0] if r0 > 0 else zrow
        nxt_row = p[r0 + nch:r0 + nch + 1] if r0 + nch < 16384 else zrow
        s0 = jnp.where(xmod != 0,
                       jnp.concatenate([prev, pc[:-1]], axis=0), 0.0)
        s2 = jnp.where(xmod != w - 1,
                       jnp.concatenate([pc[1:], nxt_row], axis=0), 0.0)
        pat = jnp.concatenate([s0, pc, s2], axis=1)
        out = jnp.dot(pat, wt, preferred_element_type=jnp.float32) + bt
        o_ref[0, :, 32 * ci:32 * (ci + 1), :] = out.T.reshape(3, 32, 128)


def _combine_kernel(inv_ref, y_ref, o_ref):
    del inv_ref
    o_ref[...] = y_ref[...]


def _wprep(cp, k):
    w = cp['w']
    cout, cin = w.shape[0], w.shape[1]
    if k == 3:
        w2 = jnp.transpose(w, (3, 2, 1, 0)).reshape(9 * cin, cout)
    else:
        w2 = w.reshape(cout, cin).T
    return [w2, cp['b'].reshape(1, cout)]


def _flat_rcan_body(p):
    seq = []
    seq += _wprep(p['head'], 3)
    for g in p['groups']:
        for blk in g['blocks']:
            seq += _wprep(blk['c1'], 3)
            seq += _wprep(blk['c2'], 3)
            seq += _wprep(blk['ca1'], 1)
            seq += _wprep(blk['ca2'], 1)
        seq += _wprep(g['conv'], 3)
    seq += _wprep(p['body_conv'], 3)
    seq += _wprep(p['up1'], 3)
    return seq


def _run_subnet(pad_idx, x_nhwc, params, cap):
    nf = params['head']['w'].shape[0]
    flat_w = _flat_rcan_body(params)
    in_specs = [pl.BlockSpec((1, 32, 32, 3), lambda i, idx: (idx[i], 0, 0, 0))]
    for wa in flat_w:
        in_specs.append(pl.BlockSpec(wa.shape,
                                     functools.partial(lambda n, i, idx: (0,) * n,
                                                       wa.ndim)))
    va = pl.pallas_call(
        _rcan_body_kernel,
        grid_spec=pltpu.PrefetchScalarGridSpec(
            num_scalar_prefetch=1,
            grid=(cap,),
            in_specs=in_specs,
            out_specs=pl.BlockSpec((1, 1024, 4 * nf), lambda i, idx: (i, 0, 0)),
        ),
        out_shape=jax.ShapeDtypeStruct((cap, 1024, 4 * nf), jnp.float32),
        compiler_params=pltpu.CompilerParams(dimension_semantics=("parallel",)),
    )(pad_idx, x_nhwc, *flat_w)

    u1 = _ps_xla(va, 32)
    wu2, bu2 = _wprep(params['up2'], 3)
    vb = pl.pallas_call(
        _up2_kernel,
        grid=(cap,),
        in_specs=[pl.BlockSpec((1, 4096, nf), lambda i: (i, 0, 0)),
                  pl.BlockSpec(wu2.shape, lambda i: (0, 0)),
                  pl.BlockSpec(bu2.shape, lambda i: (0, 0))],
        out_specs=pl.BlockSpec((1, 4096, 4 * nf), lambda i: (i, 0, 0)),
        out_shape=jax.ShapeDtypeStruct((cap, 4096, 4 * nf), jnp.float32),
        compiler_params=pltpu.CompilerParams(dimension_semantics=("parallel",)),
    )(u1, wu2, bu2)

    u2 = _ps_xla(vb, 64)
    wt, bt = _wprep(params['tail'], 3)
    return pl.pallas_call(
        _tail_kernel,
        grid=(cap,),
        in_specs=[pl.BlockSpec((1, 16384, nf), lambda i: (i, 0, 0)),
                  pl.BlockSpec(wt.shape, lambda i: (0, 0)),
                  pl.BlockSpec(bt.shape, lambda i: (0, 0))],
        out_specs=pl.BlockSpec((1, 3, 128, 128), lambda i: (i, 0, 0, 0)),
        out_shape=jax.ShapeDtypeStruct((cap, 3, 128, 128), jnp.float32),
        compiler_params=pltpu.CompilerParams(dimension_semantics=("parallel",)),
    )(u2, wt, bt)


def kernel(x, classifier_params, net1_params, net2_params, net3_params):
    b = x.shape[0]

    xc = x.reshape(b, 3, 8, 4, 8, 4).transpose(0, 2, 4, 1, 3, 5).reshape(b, 64, 48)
    cp = classifier_params
    cw1 = jnp.transpose(cp['c1']['w'], (1, 2, 3, 0)).reshape(48, 128)
    cb1 = cp['c1']['b'].reshape(1, 128)
    cw2 = cp['c2']['w'].reshape(128, 128).T
    cb2 = cp['c2']['b'].reshape(1, 128)
    fw = cp['fc_w']
    fb = cp['fc_b'].reshape(1, 3)

    pad0, pad1, pad2, inv, takes = pl.pallas_call(
        _classifier_route_kernel,
        out_shape=[
            jax.ShapeDtypeStruct((_SUBNET_CAPS[0], 1), jnp.int32),
            jax.ShapeDtypeStruct((_SUBNET_CAPS[1], 1), jnp.int32),
            jax.ShapeDtypeStruct((_SUBNET_CAPS[2], 1), jnp.int32),
            jax.ShapeDtypeStruct((b, 1), jnp.int32),
            jax.ShapeDtypeStruct((3, 1), jnp.int32),
        ],
    )(xc, cw1, cb1, cw2, cb2, fw, fb)

    x_nhwc = jnp.transpose(x, (0, 2, 3, 1))
    ys = []
    for pad_e, params, cap in ((pad0, net1_params, _SUBNET_CAPS[0]),
                               (pad1, net2_params, _SUBNET_CAPS[1]),
                               (pad2, net3_params, _SUBNET_CAPS[2])):
        ys.append(_run_subnet(pad_e.reshape(cap), x_nhwc, params, cap))

    y_ext = jnp.concatenate(ys + [jnp.zeros((1, 3, 128, 128), jnp.float32)], axis=0)
    outs = pl.pallas_call(
        _combine_kernel,
        grid_spec=pltpu.PrefetchScalarGridSpec(
            num_scalar_prefetch=1,
            grid=(b,),
            in_specs=[pl.BlockSpec((1, 3, 128, 128), lambda i, inv: (inv[i], 0, 0, 0))],
            out_specs=pl.BlockSpec((1, 3, 128, 128), lambda i, inv: (i, 0, 0, 0)),
        ),
        out_shape=jax.ShapeDtypeStruct((b, 3, 128, 128), jnp.float32),
    )(inv.reshape(b), y_ext)

    return outs, takes.reshape(3)
